# Initial kernel scaffold; baseline (speedup 1.0000x reference)
import jax
import jax.numpy as jnp
from jax.experimental import pallas as pl


def kernel(X, row_pointers, column_index, blockPartition, edgeToColumn, edgeToRow, W, attention_w):
    raise NotImplementedError("write your pallas kernel here")


# speedup vs baseline: 3.3424x; 3.3424x over previous
import functools
import jax
import jax.numpy as jnp
from jax import lax
from jax.experimental import pallas as pl
from jax.experimental.pallas import tpu as pltpu
from jax.experimental.pallas import tpu_sc as plsc

_N = 10000
_E = 320000
_D = 128
_NC = 2
_NS = 16
_EPT = _E // (_NC * _NS)
_CH = 128
_NFULL = _EPT // _CH
_TAIL = _EPT - _NFULL * _CH
_RPT = 632
_ACC_ROWS = _NS * _RPT


def _matmul(x, w):
    def mm(x_ref, w_ref, o_ref):
        o_ref[...] = jnp.dot(x_ref[...], w_ref[...],
                             preferred_element_type=jnp.float32)
    return pl.pallas_call(
        mm,
        grid=(10,),
        in_specs=[pl.BlockSpec((_N // 10, _D), lambda i: (i, 0)),
                  pl.BlockSpec((_D, _D), lambda i: (0, 0))],
        out_specs=pl.BlockSpec((_N // 10, _D), lambda i: (i, 0)),
        out_shape=jax.ShapeDtypeStruct((_N, _D), jnp.float32),
    )(x, w)


def _add_partials(parts):
    def body(p_ref, o_ref):
        o_ref[...] = p_ref[0] + p_ref[1]
    return pl.pallas_call(
        body,
        grid=(10,),
        in_specs=[pl.BlockSpec((2, _N // 10, _D), lambda i: (0, i, 0))],
        out_specs=pl.BlockSpec((_N // 10, _D), lambda i: (i, 0)),
        out_shape=jax.ShapeDtypeStruct((_N, _D), jnp.float32),
    )(parts)


@functools.partial(
    pl.kernel,
    out_type=jax.ShapeDtypeStruct((_NC * _ACC_ROWS, _D), jnp.float32),
    mesh=plsc.VectorSubcoreMesh(core_axis_name="c", subcore_axis_name="s"),
    scratch_types=[
        pltpu.VMEM((_CH,), jnp.int32),
        pltpu.VMEM((_CH,), jnp.int32),
        pltpu.VMEM((_CH, _D), jnp.float32),
        pltpu.VMEM((_TAIL,), jnp.int32),
        pltpu.VMEM((_TAIL,), jnp.int32),
        pltpu.VMEM((_TAIL, _D), jnp.float32),
        pltpu.VMEM((8, _D), jnp.float32),
        pltpu.VMEM_SHARED((_ACC_ROWS, _D), jnp.float32),
        pltpu.SemaphoreType.DMA,
    ],
)
def _spmm_round(xp_hbm, col_hbm, rid_hbm, out_hbm,
                colv, ridv, rowsv, colt, ridt, rowst, zbuf, acc, sem):
    cid = lax.axis_index("c")
    sid = lax.axis_index("s")

    def zstore(i, _):
        for j in range(_D // 16):
            zbuf[i, pl.ds(j * 16, 16)] = jnp.zeros((16,), jnp.float32)
        return 0
    lax.fori_loop(0, 8, zstore, 0)

    zbase = sid * _RPT
    def zcopy(k, _):
        pltpu.sync_copy(zbuf, acc.at[pl.ds(zbase + k * 8, 8)])
        return 0
    lax.fori_loop(0, _RPT // 8, zcopy, 0)

    plsc.subcore_barrier()

    ebase = (cid * _NS + sid) * _EPT
    def step(k, _):
        off = ebase + k * _CH
        pltpu.sync_copy(col_hbm.at[pl.ds(off, _CH)], colv)
        pltpu.sync_copy(rid_hbm.at[pl.ds(off, _CH)], ridv)
        pltpu.async_copy(xp_hbm.at[colv], rowsv, sem).wait()
        pltpu.sync_copy(rowsv, acc.at[ridv], add=True)
        return 0
    lax.fori_loop(0, _NFULL, step, 0)

    toff = ebase + _NFULL * _CH
    pltpu.sync_copy(col_hbm.at[pl.ds(toff, _TAIL)], colt)
    pltpu.sync_copy(rid_hbm.at[pl.ds(toff, _TAIL)], ridt)
    pltpu.async_copy(xp_hbm.at[colt], rowst, sem).wait()
    pltpu.sync_copy(rowst, acc.at[ridt], add=True)

    plsc.subcore_barrier()

    pltpu.sync_copy(acc.at[pl.ds(zbase, _RPT)],
                    out_hbm.at[pl.ds(cid * _ACC_ROWS + zbase, _RPT)])


def kernel(X, row_pointers, column_index, blockPartition, edgeToColumn,
           edgeToRow, W, attention_w):
    deg = row_pointers[1:] - row_pointers[:-1]
    row_ids = jnp.repeat(jnp.arange(_N, dtype=jnp.int32), deg,
                         total_repeat_length=_E)
    xp = _matmul(X, W)
    for _ in range(8):
        flat = _spmm_round(xp, column_index, row_ids)
        xp = _add_partials(flat.reshape(_NC, _ACC_ROWS, _D))
    return xp
